# baseline (speedup 1.0000x reference)
import math

import jax
import jax.numpy as jnp
from jax import lax
from jax.experimental import pallas as pl
from jax.experimental.pallas import tpu as pltpu

SCALE = 30.0
MARGIN = 0.5
COS_M = math.cos(MARGIN)
SIN_M = math.sin(MARGIN)
TH = math.cos(math.pi - MARGIN)
MM = math.sin(math.pi - MARGIN) * MARGIN

B = 1024
C = 100000

BR = 8
NSTEPS = B // BR
NBUF = 12
LA = 6


def _stream_body(lab_smem, cos_hbm, out_hbm, buf, in_sems, out_sems):
    s = pl.program_id(0)
    slot = lax.rem(s, NBUF)

    def in_copy(step, k):
        return pltpu.make_async_copy(
            cos_hbm.at[pl.ds(step * BR, BR), :], buf.at[k], in_sems.at[k])

    def out_copy(step, k):
        return pltpu.make_async_copy(
            buf.at[k], out_hbm.at[pl.ds(step * BR, BR), :], out_sems.at[k])

    @pl.when(s == 0)
    def _():
        for k in range(LA):
            in_copy(k, k).start(priority=1)

    @pl.when(s + LA < NSTEPS)
    def _():
        slot2 = lax.rem(s + LA, NBUF)

        @pl.when(s + LA >= NBUF)
        def _():
            out_copy(s + LA - NBUF, slot2).wait()

        in_copy(s + LA, slot2).start(priority=1)

    in_copy(s, slot).wait()

    lane = lax.broadcasted_iota(jnp.int32, (1, 128), 1)
    for k in range(BR):
        col = lab_smem[s * BR + k]
        base = pl.multiple_of((col // 128) * 128, 128)
        seg = buf[slot, pl.ds(k, 1), pl.ds(base, 128)]
        sine = jnp.sqrt(1.0 - seg * seg)
        phi = seg * COS_M - sine * SIN_M
        phi = jnp.where(seg > TH, phi, seg - MM)
        buf[slot, pl.ds(k, 1), pl.ds(base, 128)] = jnp.where(
            lane == (col - base), phi, seg)
    buf[slot] = buf[slot] * SCALE

    out_copy(s, slot).start()

    @pl.when(s == NSTEPS - 1)
    def _():
        for k in range(NBUF):
            step = NSTEPS - 1 - k
            out_copy(step, lax.rem(step, NBUF)).wait()


def kernel(cosine_theta_logits, label):
    lab32 = label.astype(jnp.int32)
    out = pl.pallas_call(
        _stream_body,
        grid=(NSTEPS,),
        in_specs=[
            pl.BlockSpec(memory_space=pltpu.SMEM),
            pl.BlockSpec(memory_space=pl.ANY),
        ],
        out_specs=pl.BlockSpec(memory_space=pl.ANY),
        out_shape=jax.ShapeDtypeStruct((B, C), jnp.float32),
        scratch_shapes=[
            pltpu.VMEM((NBUF, BR, C), jnp.float32),
            pltpu.SemaphoreType.DMA((NBUF,)),
            pltpu.SemaphoreType.DMA((NBUF,)),
        ],
        compiler_params=pltpu.CompilerParams(
            dimension_semantics=("arbitrary",),
        ),
    )(lab32, cosine_theta_logits)
    return out


# speedup vs baseline: 1.0063x; 1.0063x over previous
import math

import jax
import jax.numpy as jnp
from jax import lax
from jax.experimental import pallas as pl
from jax.experimental.pallas import tpu as pltpu

SCALE = 30.0
MARGIN = 0.5
COS_M = math.cos(MARGIN)
SIN_M = math.sin(MARGIN)
TH = math.cos(math.pi - MARGIN)
MM = math.sin(math.pi - MARGIN) * MARGIN

B = 1024
C = 100000

BR = 8
NSTEPS = B // BR
NBUF = 12
LA = 6


def _stream_body(lab_smem, cos_hbm, out_hbm, buf, in_sems, out_sems):
    def in_copy(step, k):
        return pltpu.make_async_copy(
            cos_hbm.at[pl.ds(step * BR, BR), :], buf.at[k], in_sems.at[k])

    def out_copy(step, k):
        return pltpu.make_async_copy(
            buf.at[k], out_hbm.at[pl.ds(step * BR, BR), :], out_sems.at[k])

    for k in range(LA):
        in_copy(k, k).start()

    lane = lax.broadcasted_iota(jnp.int32, (1, 128), 1)

    def step(s, carry):
        slot = lax.rem(s, NBUF)

        @pl.when(s + LA < NSTEPS)
        def _():
            slot2 = lax.rem(s + LA, NBUF)

            @pl.when(s + LA >= NBUF)
            def _():
                out_copy(s + LA - NBUF, slot2).wait()

            in_copy(s + LA, slot2).start()

        in_copy(s, slot).wait()

        for k in range(BR):
            col = lab_smem[s * BR + k]
            base = pl.multiple_of((col // 128) * 128, 128)
            seg = buf[slot, pl.ds(k, 1), pl.ds(base, 128)]
            sine = jnp.sqrt(1.0 - seg * seg)
            phi = seg * COS_M - sine * SIN_M
            phi = jnp.where(seg > TH, phi, seg - MM)
            buf[slot, pl.ds(k, 1), pl.ds(base, 128)] = jnp.where(
                lane == (col - base), phi, seg)
        buf[slot] = buf[slot] * SCALE

        out_copy(s, slot).start()
        return carry

    lax.fori_loop(0, NSTEPS, step, 0)

    for k in range(NBUF):
        step_id = NSTEPS - 1 - k
        out_copy(step_id, step_id % NBUF).wait()


def kernel(cosine_theta_logits, label):
    lab32 = label.astype(jnp.int32)
    out = pl.pallas_call(
        _stream_body,
        in_specs=[
            pl.BlockSpec(memory_space=pltpu.SMEM),
            pl.BlockSpec(memory_space=pl.ANY),
        ],
        out_specs=pl.BlockSpec(memory_space=pl.ANY),
        out_shape=jax.ShapeDtypeStruct((B, C), jnp.float32),
        scratch_shapes=[
            pltpu.VMEM((NBUF, BR, C), jnp.float32),
            pltpu.SemaphoreType.DMA((NBUF,)),
            pltpu.SemaphoreType.DMA((NBUF,)),
        ],
    )(lab32, cosine_theta_logits)
    return out
